# baseline (speedup 1.0000x reference)
import functools

import jax
import jax.numpy as jnp
from jax import lax
from jax.experimental import pallas as pl
from jax.experimental.pallas import tpu as pltpu
from jax.experimental.pallas import tpu_sc as plsc

NC = 2
NS = 16
NW = NC * NS
L = 16
B = 16384
D = 64
CH = B // NW
CW = 128
NCH = CH // CW
MARGIN = 1.0

_mesh = plsc.VectorSubcoreMesh(core_axis_name="c", subcore_axis_name="s")


@functools.partial(
    pl.kernel,
    mesh=_mesh,
    compiler_params=pltpu.CompilerParams(
        needs_layout_passes=False, use_tc_tiling_on_sc=False
    ),
    out_type=(
        jax.ShapeDtypeStruct((B,), jnp.float32),
        jax.ShapeDtypeStruct((B,), jnp.float32),
        jax.ShapeDtypeStruct((B,), jnp.float32),
    ),
    scratch_types=[
        pltpu.VMEM((NCH, CW), jnp.int32),
        pltpu.VMEM((NCH, CW), jnp.int32),
        pltpu.VMEM((NCH, CW), jnp.int32),
        pltpu.VMEM((NCH, CW, D), jnp.float32),
        pltpu.VMEM((NCH, CW, D), jnp.float32),
        pltpu.VMEM((NCH, CW, D), jnp.float32),
        pltpu.VMEM((CH * L,), jnp.float32),
        pltpu.VMEM((CH,), jnp.float32),
        pltpu.VMEM((CH,), jnp.float32),
        pltpu.VMEM((CH,), jnp.float32),
        pltpu.SemaphoreType.DMA,
    ],
)
def _trans_e_sc(ph, pr, pt, nh, nr, nt, ent, rel,
                loss_out, pos_out, neg_out,
                hidx, ridx, tidx, hrow, rrow, trow, psum,
                dpos, dneg, dloss, sem):
    wid = lax.axis_index("s") * NC + lax.axis_index("c")
    cbase = wid * NCH
    base = wid * CH

    def side(h_hbm, r_hbm, t_hbm, dout):
        pltpu.sync_copy(h_hbm.at[pl.ds(cbase, NCH)], hidx)
        pltpu.sync_copy(r_hbm.at[pl.ds(cbase, NCH)], ridx)
        pltpu.sync_copy(t_hbm.at[pl.ds(cbase, NCH)], tidx)
        copies = []
        for c in range(NCH):
            copies.append(pltpu.async_copy(ent.at[hidx.at[c]], hrow.at[c], sem))
            copies.append(pltpu.async_copy(rel.at[ridx.at[c]], rrow.at[c], sem))
            copies.append(pltpu.async_copy(ent.at[tidx.at[c]], trow.at[c], sem))
        for cp in copies:
            cp.wait()

        for c in range(NCH):
            def pass1(i, _, c=c):
                acc = jnp.zeros((L,), jnp.float32)
                for j in range(D // L):
                    h = hrow[c, i, pl.ds(j * L, L)]
                    r = rrow[c, i, pl.ds(j * L, L)]
                    t = trow[c, i, pl.ds(j * L, L)]
                    acc = acc + jnp.abs(h + r - t)
                psum[pl.ds((c * CW + i) * L, L)] = acc
                return 0

            lax.fori_loop(0, CW, pass1, 0)

        def pass2(g, _):
            rows = (g * L + lax.iota(jnp.int32, L)) * L
            acc = jnp.zeros((L,), jnp.float32)
            for j in range(L):
                acc = acc + plsc.load_gather(psum, [rows + j])
            dout[pl.ds(g * L, L)] = acc
            return 0

        lax.fori_loop(0, CH // L, pass2, 0)

    side(ph, pr, pt, dpos)
    side(nh, nr, nt, dneg)

    def loss_body(g, _):
        p = dpos[pl.ds(g * L, L)]
        n = dneg[pl.ds(g * L, L)]
        dloss[pl.ds(g * L, L)] = jnp.maximum(p - n + MARGIN, 0.0)
        return 0

    lax.fori_loop(0, CH // L, loss_body, 0)

    pltpu.sync_copy(dloss, loss_out.at[pl.ds(base, CH)])
    pltpu.sync_copy(dpos, pos_out.at[pl.ds(base, CH)])
    pltpu.sync_copy(dneg, neg_out.at[pl.ds(base, CH)])


def kernel(positive_triplets, negative_triplets, entities_emb, relations_emb):
    p32 = positive_triplets.astype(jnp.int32)
    n32 = negative_triplets.astype(jnp.int32)
    ph = p32[:, 0].reshape(B // CW, CW)
    pr = p32[:, 1].reshape(B // CW, CW)
    pt = p32[:, 2].reshape(B // CW, CW)
    nh = n32[:, 0].reshape(B // CW, CW)
    nr = n32[:, 1].reshape(B // CW, CW)
    nt = n32[:, 2].reshape(B // CW, CW)
    return _trans_e_sc(ph, pr, pt, nh, nr, nt, entities_emb, relations_emb)
